# baseline (speedup 1.0000x reference)
import dataclasses
import functools

import jax
import jax.numpy as jnp
from jax import lax
from jax.experimental import pallas as pl
from jax.experimental.pallas import tpu as pltpu
from jax.experimental.pallas import tpu_sc as plsc

N = 10000
E = 320000
D = 128
ALPHA = 0.2
SCALE = 1.0 / (D ** 0.5)

NC = 2
NS = 16
LANES = 16
NW = NC * NS
EPW = E // NW
CHUNK = 80
NCHUNK = EPW // CHUNK
NPAD = 10240
_GATHER_DNUMS = lax.GatherDimensionNumbers(
    offset_dims=(), collapsed_slice_dims=(0,), start_index_map=(0,))


def _lane_bcast(vec, lane):
    idx = jnp.full((LANES, 1), lane, jnp.int32)
    return lax.gather(vec, idx, _GATHER_DNUMS, slice_sizes=(1,),
                      mode=lax.GatherScatterMode.PROMISE_IN_BOUNDS)
PERT = NPAD // NS
RPT = NPAD // NS

_mesh = plsc.VectorSubcoreMesh(core_axis_name="c", subcore_axis_name="s")

_sc_params = pltpu.CompilerParams()
if "needs_layout_passes" in pltpu.CompilerParams.__dataclass_fields__:
    _sc_params = dataclasses.replace(_sc_params, needs_layout_passes=False)


def _qkv_body(x_ref, wq_ref, wk_ref, wv_ref, q_ref, k_ref, v_ref):
    x = x_ref[...]
    dn = (((1,), (1,)), ((), ()))
    q_ref[...] = lax.dot_general(x, wq_ref[...] * SCALE, dn,
                                 preferred_element_type=jnp.float32)
    k_ref[...] = lax.dot_general(x, wk_ref[...], dn,
                                 preferred_element_type=jnp.float32)
    v_ref[...] = lax.dot_general(x, wv_ref[...], dn,
                                 preferred_element_type=jnp.float32)


def _qkv(residual, wq, wk, wv):
    blk = 2000
    grid = (N // blk,)
    out = jax.ShapeDtypeStruct((N, D), jnp.float32)
    return pl.pallas_call(
        _qkv_body,
        grid=grid,
        in_specs=[
            pl.BlockSpec((blk, D), lambda i: (i, 0)),
            pl.BlockSpec((D, D), lambda i: (0, 0)),
            pl.BlockSpec((D, D), lambda i: (0, 0)),
            pl.BlockSpec((D, D), lambda i: (0, 0)),
        ],
        out_specs=[
            pl.BlockSpec((blk, D), lambda i: (i, 0)),
            pl.BlockSpec((blk, D), lambda i: (i, 0)),
            pl.BlockSpec((blk, D), lambda i: (i, 0)),
        ],
        out_shape=[out, out, out],
    )(residual, wq, wk, wv)


def _edges_body(q_hbm, k_hbm, src_hbm, dst_hbm, exps_hbm, denom_hbm,
                idx_s0, idx_s1, idx_d0, idx_d1, qr0, qr1, kr0, kr1,
                esc0, esc1, stage, dloc, dred, slab,
                sem_i0, sem_i1, sem_g0, sem_g1, sem_e0, sem_e1):
    idx_s = [idx_s0, idx_s1]
    idx_d = [idx_d0, idx_d1]
    qr = [qr0, qr1]
    kr = [kr0, kr1]
    esc = [esc0, esc1]
    sem_i = [sem_i0, sem_i1]
    sem_g = [sem_g0, sem_g1]
    sem_e = [sem_e0, sem_e1]

    cid = lax.axis_index("c")
    sid = lax.axis_index("s")
    wid = cid * NS + sid
    base0 = wid * EPW
    lane_iota = lax.iota(jnp.int32, LANES)

    @pl.loop(0, NPAD, step=LANES)
    def _zero(i):
        dloc[pl.ds(i, LANES)] = jnp.zeros((LANES,), jnp.float32)

    def issue_idx(c, b):
        base = base0 + c * CHUNK
        pltpu.async_copy(src_hbm.at[pl.ds(base, CHUNK)], idx_s[b], sem_i[b])
        pltpu.async_copy(dst_hbm.at[pl.ds(base, CHUNK)], idx_d[b], sem_i[b])

    def drain_idx(b):
        pltpu.make_async_copy(src_hbm.at[pl.ds(0, CHUNK)], idx_s[b],
                              sem_i[b]).wait()
        pltpu.make_async_copy(dst_hbm.at[pl.ds(0, CHUNK)], idx_d[b],
                              sem_i[b]).wait()

    def issue_gather(b):
        pltpu.async_copy(q_hbm.at[idx_s[b]], qr[b], sem_g[b])
        pltpu.async_copy(k_hbm.at[idx_d[b]], kr[b], sem_g[b])

    def drain_gather(b):
        pltpu.make_async_copy(q_hbm.at[idx_s[b]], qr[b], sem_g[b]).wait()
        pltpu.make_async_copy(k_hbm.at[idx_d[b]], kr[b], sem_g[b]).wait()

    def drain_esc(b):
        pltpu.make_async_copy(esc[b], exps_hbm.at[pl.ds(base0, CHUNK)],
                              sem_e[b]).wait()

    def body(c, b):
        drain_gather(b)

        @pl.when(c + 1 < NCHUNK)
        def _():
            drain_idx(1 - b)
            issue_gather(1 - b)

        @pl.when(c >= 2)
        def _():
            drain_esc(b)

        @pl.loop(0, CHUNK, step=LANES)
        def _dotgrp(g):
            for l in range(LANES):
                e = g + l
                acc = qr[b][e, pl.ds(0, LANES)] * kr[b][e, pl.ds(0, LANES)]
                for j in range(1, D // LANES):
                    acc = acc + (qr[b][e, pl.ds(j * LANES, LANES)]
                                 * kr[b][e, pl.ds(j * LANES, LANES)])
                stage[pl.ds(l * LANES, LANES)] = plsc.cumsum(acc)
            sv = plsc.load_gather(stage, [lane_iota * LANES + (LANES - 1)])
            sv = jnp.maximum(sv, ALPHA * sv)
            ev = jnp.exp(sv)
            esc[b][pl.ds(g, LANES)] = ev
            plsc.addupdate_scatter(dloc, [idx_s[b][pl.ds(g, LANES)]], ev)

        pltpu.async_copy(esc[b], exps_hbm.at[pl.ds(base0 + c * CHUNK, CHUNK)],
                         sem_e[b])

        @pl.when(c + 2 < NCHUNK)
        def _():
            issue_idx(c + 2, b)

    issue_idx(0, 0)
    issue_idx(1, 1)
    drain_idx(0)
    issue_gather(0)

    @pl.loop(0, NCHUNK - 1, step=2)
    def _main(c):
        body(c, 0)
        body(c + 1, 1)

    body(NCHUNK - 1, 0)
    drain_esc(0)
    drain_esc(1)

    pltpu.sync_copy(dloc, slab.at[sid])
    plsc.subcore_barrier()
    for r in range(NS):
        pltpu.sync_copy(slab.at[r, pl.ds(sid * PERT, PERT)], dred.at[r])

    @pl.loop(0, PERT, step=LANES)
    def _red(i):
        acc = dred[0, pl.ds(i, LANES)]
        for r in range(1, NS):
            acc = acc + dred[r, pl.ds(i, LANES)]
        dred[0, pl.ds(i, LANES)] = acc

    pltpu.sync_copy(dred.at[0], denom_hbm.at[cid, pl.ds(sid * PERT, PERT)])


def _edge_scores(q, k, src, dst):
    kern = pl.kernel(
        _edges_body,
        out_type=(
            jax.ShapeDtypeStruct((E,), jnp.float32),
            jax.ShapeDtypeStruct((NC, NPAD), jnp.float32),
        ),
        mesh=_mesh,
        scratch_types=[
            pltpu.VMEM((CHUNK,), jnp.int32),
            pltpu.VMEM((CHUNK,), jnp.int32),
            pltpu.VMEM((CHUNK,), jnp.int32),
            pltpu.VMEM((CHUNK,), jnp.int32),
            pltpu.VMEM((CHUNK, D), jnp.float32),
            pltpu.VMEM((CHUNK, D), jnp.float32),
            pltpu.VMEM((CHUNK, D), jnp.float32),
            pltpu.VMEM((CHUNK, D), jnp.float32),
            pltpu.VMEM((CHUNK,), jnp.float32),
            pltpu.VMEM((CHUNK,), jnp.float32),
            pltpu.VMEM((LANES * LANES,), jnp.float32),
            pltpu.VMEM((NPAD,), jnp.float32),
            pltpu.VMEM((NS, PERT), jnp.float32),
            pltpu.VMEM_SHARED((NS, NPAD), jnp.float32),
            pltpu.SemaphoreType.DMA,
            pltpu.SemaphoreType.DMA,
            pltpu.SemaphoreType.DMA,
            pltpu.SemaphoreType.DMA,
            pltpu.SemaphoreType.DMA,
            pltpu.SemaphoreType.DMA,
        ],
        compiler_params=_sc_params,
    )
    return kern(q, k, src, dst)


def _rdenom_body(d_ref, o_ref):
    o_ref[...] = 1.0 / (d_ref[0, :] + d_ref[1, :] + 1e-16)


def _rdenom(denom2):
    return pl.pallas_call(
        _rdenom_body,
        grid=(1,),
        in_specs=[pl.BlockSpec((NC, NPAD), lambda i: (0, 0))],
        out_specs=pl.BlockSpec((NPAD,), lambda i: (0,)),
        out_shape=jax.ShapeDtypeStruct((NPAD,), jnp.float32),
    )(denom2)


def _aggr_body(v_hbm, src_hbm, dst_hbm, exps_hbm, rdenom_hbm, outp_hbm,
               idx_s0, idx_s1, idx_s2, idx_d0, idx_d1, idx_d2,
               vr0, vr1, vr2, esc0, esc1, esc2, wv, dloc, out_sh,
               sem_i0, sem_i1, sem_i2, sem_g0, sem_g1, sem_g2,
               sem_s0, sem_s1, sem_s2):
    idx_s = [idx_s0, idx_s1, idx_s2]
    idx_d = [idx_d0, idx_d1, idx_d2]
    vr = [vr0, vr1, vr2]
    esc = [esc0, esc1, esc2]
    sem_i = [sem_i0, sem_i1, sem_i2]
    sem_g = [sem_g0, sem_g1, sem_g2]
    sem_s = [sem_s0, sem_s1, sem_s2]

    cid = lax.axis_index("c")
    sid = lax.axis_index("s")
    wid = cid * NS + sid
    base0 = wid * EPW

    pltpu.sync_copy(rdenom_hbm, dloc)

    @pl.loop(0, CHUNK)
    def _zv(r):
        for j in range(D // LANES):
            vr0[r, pl.ds(j * LANES, LANES)] = jnp.zeros((LANES,),
                                                        jnp.float32)

    for bk in range(RPT // CHUNK):
        pltpu.sync_copy(vr0, out_sh.at[pl.ds(sid * RPT + bk * CHUNK, CHUNK)])

    def issue_idx(c, b):
        base = base0 + c * CHUNK
        pltpu.async_copy(src_hbm.at[pl.ds(base, CHUNK)], idx_s[b], sem_i[b])
        pltpu.async_copy(dst_hbm.at[pl.ds(base, CHUNK)], idx_d[b], sem_i[b])
        pltpu.async_copy(exps_hbm.at[pl.ds(base, CHUNK)], esc[b], sem_i[b])

    def drain_idx(b):
        pltpu.make_async_copy(src_hbm.at[pl.ds(0, CHUNK)], idx_s[b],
                              sem_i[b]).wait()
        pltpu.make_async_copy(dst_hbm.at[pl.ds(0, CHUNK)], idx_d[b],
                              sem_i[b]).wait()
        pltpu.make_async_copy(exps_hbm.at[pl.ds(0, CHUNK)], esc[b],
                              sem_i[b]).wait()

    def issue_gather(b):
        pltpu.async_copy(v_hbm.at[idx_d[b]], vr[b], sem_g[b])

    def drain_gather(b):
        pltpu.make_async_copy(v_hbm.at[idx_d[b]], vr[b], sem_g[b]).wait()

    def drain_scatter(b):
        pltpu.make_async_copy(vr[b], out_sh.at[idx_s[b]], sem_s[b]).wait()

    def body(c, b):
        b1 = (b + 1) % 3
        b2 = (b + 2) % 3
        drain_gather(b)

        @pl.loop(0, CHUNK, step=LANES)
        def _wts(g):
            dv = plsc.load_gather(dloc, [idx_s[b][pl.ds(g, LANES)]])
            wv[pl.ds(g, LANES)] = esc[b][pl.ds(g, LANES)] * dv

        @pl.loop(0, CHUNK, step=LANES)
        def _scalerow(g):
            wvec = wv[pl.ds(g, LANES)]
            for l in range(LANES):
                e = g + l
                wb = _lane_bcast(wvec, l)
                for j in range(D // LANES):
                    vr[b][e, pl.ds(j * LANES, LANES)] = (
                        vr[b][e, pl.ds(j * LANES, LANES)] * wb)

        @pl.when(c >= 1)
        def _():
            drain_scatter(b2)

        pltpu.async_copy(vr[b], out_sh.at[idx_s[b]], sem_s[b], add=True)

        @pl.when(c + 1 < NCHUNK)
        def _():
            drain_idx(b1)
            issue_gather(b1)

        @pl.when(c + 2 < NCHUNK)
        def _():
            issue_idx(c + 2, b2)

    issue_idx(0, 0)
    issue_idx(1, 1)
    drain_idx(0)
    issue_gather(0)
    plsc.subcore_barrier()

    @pl.loop(0, NCHUNK - 2, step=3)
    def _main(c):
        body(c, 0)
        body(c + 1, 1)
        body(c + 2, 2)

    body(NCHUNK - 2, 0)
    body(NCHUNK - 1, 1)
    drain_scatter(1)

    plsc.subcore_barrier()
    pltpu.sync_copy(out_sh.at[pl.ds(sid * RPT, RPT)],
                    outp_hbm.at[cid, pl.ds(sid * RPT, RPT)])


def _aggregate(v, src, dst, exps, rdenom):
    kern = pl.kernel(
        _aggr_body,
        out_type=jax.ShapeDtypeStruct((NC, NPAD, D), jnp.float32),
        mesh=_mesh,
        scratch_types=(
            [pltpu.VMEM((CHUNK,), jnp.int32)] * 6
            + [pltpu.VMEM((CHUNK, D), jnp.float32)] * 3
            + [pltpu.VMEM((CHUNK,), jnp.float32)] * 3
            + [
                pltpu.VMEM((CHUNK,), jnp.float32),
                pltpu.VMEM((NPAD,), jnp.float32),
                pltpu.VMEM_SHARED((NPAD, D), jnp.float32),
            ]
            + [pltpu.SemaphoreType.DMA] * 9
        ),
        compiler_params=_sc_params,
    )
    return kern(v, src, dst, exps, rdenom)


def _add_body(a_ref, b_ref, o_ref):
    o_ref[...] = a_ref[...] + b_ref[...]


def _combine(p0, p1):
    blk = 2000
    return pl.pallas_call(
        _add_body,
        grid=(N // blk,),
        in_specs=[
            pl.BlockSpec((blk, D), lambda i: (i, 0)),
            pl.BlockSpec((blk, D), lambda i: (i, 0)),
        ],
        out_specs=pl.BlockSpec((blk, D), lambda i: (i, 0)),
        out_shape=jax.ShapeDtypeStruct((N, D), jnp.float32),
    )(p0, p1)


def kernel(residual, edge_index, Wq, Wk, Wv):
    src = edge_index[0]
    dst = edge_index[1]
    q, k, v = _qkv(residual, Wq, Wk, Wv)
    exps, denom2 = _edge_scores(q, k, src, dst)
    outp = _aggregate(v, src, dst, exps, _rdenom(denom2))
    return _combine(outp[0, :N], outp[1, :N])


# speedup vs baseline: 1.0985x; 1.0985x over previous
import dataclasses
import functools

import jax
import jax.numpy as jnp
from jax import lax
from jax.experimental import pallas as pl
from jax.experimental.pallas import tpu as pltpu
from jax.experimental.pallas import tpu_sc as plsc

N = 10000
E = 320000
D = 128
ALPHA = 0.2
SCALE = 1.0 / (D ** 0.5)

NC = 2
NS = 16
LANES = 16
NW = NC * NS
EPW = E // NW
CHUNK = 80
NCHUNK = EPW // CHUNK
NPAD = 10240
_GATHER_DNUMS = lax.GatherDimensionNumbers(
    offset_dims=(), collapsed_slice_dims=(0,), start_index_map=(0,))


def _lane_bcast(vec, lane):
    idx = jnp.full((LANES, 1), lane, jnp.int32)
    return lax.gather(vec, idx, _GATHER_DNUMS, slice_sizes=(1,),
                      mode=lax.GatherScatterMode.PROMISE_IN_BOUNDS)
PERT = NPAD // NS
RPT = NPAD // NS

_mesh = plsc.VectorSubcoreMesh(core_axis_name="c", subcore_axis_name="s")

_sc_params = pltpu.CompilerParams()
if "needs_layout_passes" in pltpu.CompilerParams.__dataclass_fields__:
    _sc_params = dataclasses.replace(_sc_params, needs_layout_passes=False)


def _qkv_body(x_ref, wq_ref, wk_ref, wv_ref, q_ref, k_ref, v_ref):
    x = x_ref[...]
    dn = (((1,), (1,)), ((), ()))
    q_ref[...] = lax.dot_general(x, wq_ref[...] * SCALE, dn,
                                 preferred_element_type=jnp.float32)
    k_ref[...] = lax.dot_general(x, wk_ref[...], dn,
                                 preferred_element_type=jnp.float32)
    v_ref[...] = lax.dot_general(x, wv_ref[...], dn,
                                 preferred_element_type=jnp.float32)


def _qkv(residual, wq, wk, wv):
    blk = 2000
    grid = (N // blk,)
    out = jax.ShapeDtypeStruct((N, D), jnp.float32)
    return pl.pallas_call(
        _qkv_body,
        grid=grid,
        in_specs=[
            pl.BlockSpec((blk, D), lambda i: (i, 0)),
            pl.BlockSpec((D, D), lambda i: (0, 0)),
            pl.BlockSpec((D, D), lambda i: (0, 0)),
            pl.BlockSpec((D, D), lambda i: (0, 0)),
        ],
        out_specs=[
            pl.BlockSpec((blk, D), lambda i: (i, 0)),
            pl.BlockSpec((blk, D), lambda i: (i, 0)),
            pl.BlockSpec((blk, D), lambda i: (i, 0)),
        ],
        out_shape=[out, out, out],
    )(residual, wq, wk, wv)


def _edges_body(q_hbm, k_hbm, src_hbm, dst_hbm, exps_hbm, denom_hbm,
                idx_s0, idx_s1, idx_d0, idx_d1, qr0, qr1, kr0, kr1,
                esc0, esc1, stage, dloc, dred, slab,
                sem_i0, sem_i1, sem_g0, sem_g1, sem_e0, sem_e1):
    idx_s = [idx_s0, idx_s1]
    idx_d = [idx_d0, idx_d1]
    qr = [qr0, qr1]
    kr = [kr0, kr1]
    esc = [esc0, esc1]
    sem_i = [sem_i0, sem_i1]
    sem_g = [sem_g0, sem_g1]
    sem_e = [sem_e0, sem_e1]

    cid = lax.axis_index("c")
    sid = lax.axis_index("s")
    wid = cid * NS + sid
    base0 = wid * EPW
    lane_iota = lax.iota(jnp.int32, LANES)

    @pl.loop(0, NPAD, step=LANES)
    def _zero(i):
        dloc[pl.ds(i, LANES)] = jnp.zeros((LANES,), jnp.float32)

    def issue_idx(c, b):
        base = base0 + c * CHUNK
        pltpu.async_copy(src_hbm.at[pl.ds(base, CHUNK)], idx_s[b], sem_i[b])
        pltpu.async_copy(dst_hbm.at[pl.ds(base, CHUNK)], idx_d[b], sem_i[b])

    def drain_idx(b):
        pltpu.make_async_copy(src_hbm.at[pl.ds(0, CHUNK)], idx_s[b],
                              sem_i[b]).wait()
        pltpu.make_async_copy(dst_hbm.at[pl.ds(0, CHUNK)], idx_d[b],
                              sem_i[b]).wait()

    def issue_gather(b):
        pltpu.async_copy(q_hbm.at[idx_s[b]], qr[b], sem_g[b])
        pltpu.async_copy(k_hbm.at[idx_d[b]], kr[b], sem_g[b])

    def drain_gather(b):
        pltpu.make_async_copy(q_hbm.at[idx_s[b]], qr[b], sem_g[b]).wait()
        pltpu.make_async_copy(k_hbm.at[idx_d[b]], kr[b], sem_g[b]).wait()

    def drain_esc(b):
        pltpu.make_async_copy(esc[b], exps_hbm.at[pl.ds(base0, CHUNK)],
                              sem_e[b]).wait()

    def body(c, b):
        drain_gather(b)

        @pl.when(c + 1 < NCHUNK)
        def _():
            drain_idx(1 - b)
            issue_gather(1 - b)

        @pl.when(c >= 2)
        def _():
            drain_esc(b)

        @pl.loop(0, CHUNK, step=LANES)
        def _dotgrp(g):
            for l in range(LANES):
                e = g + l
                acc = qr[b][e, pl.ds(0, LANES)] * kr[b][e, pl.ds(0, LANES)]
                for j in range(1, D // LANES):
                    acc = acc + (qr[b][e, pl.ds(j * LANES, LANES)]
                                 * kr[b][e, pl.ds(j * LANES, LANES)])
                stage[pl.ds(l * LANES, LANES)] = plsc.cumsum(acc)
            sv = plsc.load_gather(stage, [lane_iota * LANES + (LANES - 1)])
            sv = jnp.maximum(sv, ALPHA * sv)
            ev = jnp.exp(sv)
            esc[b][pl.ds(g, LANES)] = ev
            plsc.addupdate_scatter(dloc, [idx_s[b][pl.ds(g, LANES)]], ev)

        pltpu.async_copy(esc[b], exps_hbm.at[pl.ds(base0 + c * CHUNK, CHUNK)],
                         sem_e[b])

        @pl.when(c + 2 < NCHUNK)
        def _():
            issue_idx(c + 2, b)

    issue_idx(0, 0)
    issue_idx(1, 1)
    drain_idx(0)
    issue_gather(0)

    @pl.loop(0, NCHUNK - 1, step=2)
    def _main(c):
        body(c, 0)
        body(c + 1, 1)

    body(NCHUNK - 1, 0)
    drain_esc(0)
    drain_esc(1)

    pltpu.sync_copy(dloc, slab.at[sid])
    plsc.subcore_barrier()
    for r in range(NS):
        pltpu.sync_copy(slab.at[r, pl.ds(sid * PERT, PERT)], dred.at[r])

    @pl.loop(0, PERT, step=LANES)
    def _red(i):
        acc = dred[0, pl.ds(i, LANES)]
        for r in range(1, NS):
            acc = acc + dred[r, pl.ds(i, LANES)]
        dred[0, pl.ds(i, LANES)] = acc

    pltpu.sync_copy(dred.at[0], denom_hbm.at[cid, pl.ds(sid * PERT, PERT)])


def _edge_scores(q, k, src, dst):
    kern = pl.kernel(
        _edges_body,
        out_type=(
            jax.ShapeDtypeStruct((E,), jnp.float32),
            jax.ShapeDtypeStruct((NC, NPAD), jnp.float32),
        ),
        mesh=_mesh,
        scratch_types=[
            pltpu.VMEM((CHUNK,), jnp.int32),
            pltpu.VMEM((CHUNK,), jnp.int32),
            pltpu.VMEM((CHUNK,), jnp.int32),
            pltpu.VMEM((CHUNK,), jnp.int32),
            pltpu.VMEM((CHUNK, D), jnp.float32),
            pltpu.VMEM((CHUNK, D), jnp.float32),
            pltpu.VMEM((CHUNK, D), jnp.float32),
            pltpu.VMEM((CHUNK, D), jnp.float32),
            pltpu.VMEM((CHUNK,), jnp.float32),
            pltpu.VMEM((CHUNK,), jnp.float32),
            pltpu.VMEM((LANES * LANES,), jnp.float32),
            pltpu.VMEM((NPAD,), jnp.float32),
            pltpu.VMEM((NS, PERT), jnp.float32),
            pltpu.VMEM_SHARED((NS, NPAD), jnp.float32),
            pltpu.SemaphoreType.DMA,
            pltpu.SemaphoreType.DMA,
            pltpu.SemaphoreType.DMA,
            pltpu.SemaphoreType.DMA,
            pltpu.SemaphoreType.DMA,
            pltpu.SemaphoreType.DMA,
        ],
        compiler_params=_sc_params,
    )
    return kern(q, k, src, dst)


def _rdenom_body(d_ref, o_ref):
    o_ref[...] = 1.0 / (d_ref[0, :] + d_ref[1, :] + 1e-16)


def _rdenom(denom2):
    return pl.pallas_call(
        _rdenom_body,
        grid=(1,),
        in_specs=[pl.BlockSpec((NC, NPAD), lambda i: (0, 0))],
        out_specs=pl.BlockSpec((NPAD,), lambda i: (0,)),
        out_shape=jax.ShapeDtypeStruct((NPAD,), jnp.float32),
    )(denom2)


def _aggr_body(v_hbm, src_hbm, dst_hbm, exps_hbm, rdenom_hbm, outp_hbm,
               idx_s0, idx_s1, idx_s2, idx_d0, idx_d1, idx_d2,
               vr0, vr1, vr2, esc0, esc1, esc2, wv, dloc, out_sh,
               sem_i0, sem_i1, sem_i2, sem_g0, sem_g1, sem_g2,
               sem_s0, sem_s1, sem_s2):
    idx_s = [idx_s0, idx_s1, idx_s2]
    idx_d = [idx_d0, idx_d1, idx_d2]
    vr = [vr0, vr1, vr2]
    esc = [esc0, esc1, esc2]
    sem_i = [sem_i0, sem_i1, sem_i2]
    sem_g = [sem_g0, sem_g1, sem_g2]
    sem_s = [sem_s0, sem_s1, sem_s2]

    cid = lax.axis_index("c")
    sid = lax.axis_index("s")
    wid = cid * NS + sid
    base0 = wid * EPW

    pltpu.sync_copy(rdenom_hbm, dloc)

    @pl.loop(0, CHUNK)
    def _zv(r):
        for j in range(D // LANES):
            vr0[r, pl.ds(j * LANES, LANES)] = jnp.zeros((LANES,),
                                                        jnp.float32)

    for bk in range(RPT // CHUNK):
        pltpu.sync_copy(vr0, out_sh.at[pl.ds(sid * RPT + bk * CHUNK, CHUNK)])

    def issue_idx(c, b):
        base = base0 + c * CHUNK
        pltpu.async_copy(src_hbm.at[pl.ds(base, CHUNK)], idx_s[b], sem_i[b])
        pltpu.async_copy(dst_hbm.at[pl.ds(base, CHUNK)], idx_d[b], sem_i[b])
        pltpu.async_copy(exps_hbm.at[pl.ds(base, CHUNK)], esc[b], sem_i[b])

    def drain_idx(b):
        pltpu.make_async_copy(src_hbm.at[pl.ds(0, CHUNK)], idx_s[b],
                              sem_i[b]).wait()
        pltpu.make_async_copy(dst_hbm.at[pl.ds(0, CHUNK)], idx_d[b],
                              sem_i[b]).wait()
        pltpu.make_async_copy(exps_hbm.at[pl.ds(0, CHUNK)], esc[b],
                              sem_i[b]).wait()

    def issue_gather(b):
        pltpu.async_copy(v_hbm.at[idx_d[b]], vr[b], sem_g[b])

    def drain_gather(b):
        pltpu.make_async_copy(v_hbm.at[idx_d[b]], vr[b], sem_g[b]).wait()

    def drain_scatter(b):
        pltpu.make_async_copy(vr[b], out_sh.at[idx_s[b]], sem_s[b]).wait()

    def body(c, b):
        b1 = (b + 1) % 3
        b2 = (b + 2) % 3
        drain_gather(b)

        @pl.when(c + 1 < NCHUNK)
        def _():
            drain_idx(b1)
            issue_gather(b1)

        @pl.loop(0, CHUNK, step=LANES)
        def _wts(g):
            dv = plsc.load_gather(dloc, [idx_s[b][pl.ds(g, LANES)]])
            wv[pl.ds(g, LANES)] = esc[b][pl.ds(g, LANES)] * dv

        @pl.loop(0, CHUNK, step=LANES)
        def _scalerow(g):
            wvec = wv[pl.ds(g, LANES)]
            for l in range(LANES):
                e = g + l
                wb = _lane_bcast(wvec, l)
                for j in range(D // LANES):
                    vr[b][e, pl.ds(j * LANES, LANES)] = (
                        vr[b][e, pl.ds(j * LANES, LANES)] * wb)

        @pl.when(c >= 1)
        def _():
            drain_scatter(b2)

        pltpu.async_copy(vr[b], out_sh.at[idx_s[b]], sem_s[b], add=True)

        @pl.when(c + 2 < NCHUNK)
        def _():
            issue_idx(c + 2, b2)

    issue_idx(0, 0)
    issue_idx(1, 1)
    drain_idx(0)
    issue_gather(0)
    plsc.subcore_barrier()

    @pl.loop(0, NCHUNK - 2, step=3)
    def _main(c):
        body(c, 0)
        body(c + 1, 1)
        body(c + 2, 2)

    body(NCHUNK - 2, 0)
    body(NCHUNK - 1, 1)
    drain_scatter(1)

    plsc.subcore_barrier()
    pltpu.sync_copy(out_sh.at[pl.ds(sid * RPT, RPT)],
                    outp_hbm.at[cid, pl.ds(sid * RPT, RPT)])


def _aggregate(v, src, dst, exps, rdenom):
    kern = pl.kernel(
        _aggr_body,
        out_type=jax.ShapeDtypeStruct((NC, NPAD, D), jnp.float32),
        mesh=_mesh,
        scratch_types=(
            [pltpu.VMEM((CHUNK,), jnp.int32)] * 6
            + [pltpu.VMEM((CHUNK, D), jnp.float32)] * 3
            + [pltpu.VMEM((CHUNK,), jnp.float32)] * 3
            + [
                pltpu.VMEM((CHUNK,), jnp.float32),
                pltpu.VMEM((NPAD,), jnp.float32),
                pltpu.VMEM_SHARED((NPAD, D), jnp.float32),
            ]
            + [pltpu.SemaphoreType.DMA] * 9
        ),
        compiler_params=_sc_params,
    )
    return kern(v, src, dst, exps, rdenom)


def _add_body(a_ref, b_ref, o_ref):
    o_ref[...] = a_ref[...] + b_ref[...]


def _combine(p0, p1):
    blk = 2000
    return pl.pallas_call(
        _add_body,
        grid=(N // blk,),
        in_specs=[
            pl.BlockSpec((blk, D), lambda i: (i, 0)),
            pl.BlockSpec((blk, D), lambda i: (i, 0)),
        ],
        out_specs=pl.BlockSpec((blk, D), lambda i: (i, 0)),
        out_shape=jax.ShapeDtypeStruct((N, D), jnp.float32),
    )(p0, p1)


def kernel(residual, edge_index, Wq, Wk, Wv):
    src = edge_index[0]
    dst = edge_index[1]
    q, k, v = _qkv(residual, Wq, Wk, Wv)
    exps, denom2 = _edge_scores(q, k, src, dst)
    outp = _aggregate(v, src, dst, exps, _rdenom(denom2))
    return _combine(outp[0, :N], outp[1, :N])
